# baseline (speedup 1.0000x reference)
import jax
import jax.numpy as jnp
from jax import lax
from jax.experimental import pallas as pl
from jax.experimental.pallas import tpu as pltpu
from jax.experimental.pallas import tpu_sc as plsc

N_NODES = 10000
DIM = 128
WORDS = DIM // 2
ROW = 2 * WORDS
N_ROWS = N_NODES // 2
N_EDGES = 320000
LANES = 16

_info = plsc.get_sparse_core_info()
NC = _info.num_cores
NS = _info.num_subcores
NW = NC * NS

EDGES_PER_W = N_EDGES // NW
CHUNK = 80
ROUNDS = EDGES_PER_W // CHUNK
GROUPS = CHUNK // LANES

STRIPE = (N_ROWS // NS) // 8 * 8
TAIL = N_ROWS - STRIPE * NS


def _body(z_hbm, src_hbm, dst_hbm, out_hbm,
          zspm, sidx, didx,
          gidx_sa, gidx_da, gidx_sb, gidx_db,
          srows_a, drows_a, srows_b, drows_b, ps, outv,
          sem_sa, sem_da, sem_sb, sem_db):
    sid = lax.axis_index("s")
    wid = sid * NC + lax.axis_index("c")
    base_w = wid * EDGES_PER_W

    lanes_i = lax.iota(jnp.int32, LANES)

    sbase = pl.multiple_of(sid * STRIPE, 8)
    pltpu.sync_copy(z_hbm.at[pl.ds(sbase, STRIPE), :],
                    zspm.at[pl.ds(sbase, STRIPE), :])
    pltpu.sync_copy(z_hbm.at[pl.ds(STRIPE * NS, TAIL), :],
                    zspm.at[pl.ds(STRIPE * NS, TAIL), :])

    pltpu.sync_copy(src_hbm.at[wid], sidx)
    pltpu.sync_copy(dst_hbm.at[wid], didx)

    plsc.subcore_barrier()

    def fire(r, gidx_s, gidx_d, srows, drows, sem_s, sem_d):
        for j in range(CHUNK // LANES):
            sl = pl.ds(j * LANES, LANES)
            gidx_s[sl] = lax.shift_right_logical(sidx[r, sl], 1)
            gidx_d[sl] = lax.shift_right_logical(didx[r, sl], 1)
        pltpu.async_copy(zspm.at[gidx_s], srows, sem_s)
        pltpu.async_copy(zspm.at[gidx_d], drows, sem_d)

    def drain(gidx_s, gidx_d, srows, drows, sem_s, sem_d):
        pltpu.make_async_copy(zspm.at[gidx_s], srows, sem_s).wait()
        pltpu.make_async_copy(zspm.at[gidx_d], drows, sem_d).wait()

    def compute(r, srows, drows):
        def group_body(g, carry):
            e0 = g * LANES
            soffv = (sidx[r, pl.ds(e0, LANES)] & 1) * WORDS
            doffv = (didx[r, pl.ds(e0, LANES)] & 1) * WORDS
            for e in range(LANES):
                soff = soffv[e]
                doff = doffv[e]
                p = None
                for j in range(WORDS // LANES):
                    sw = srows[e0 + e, pl.ds(soff + j * LANES, LANES)]
                    dw = drows[e0 + e, pl.ds(doff + j * LANES, LANES)]
                    pb = (plsc.bitcast(sw, jnp.bfloat16)
                          * plsc.bitcast(dw, jnp.bfloat16))
                    plo, phi = plsc.unpack(
                        pb, format=plsc.PackFormat.INTERLEAVED)
                    p = plo + phi if p is None else p + plo + phi
                ps[pl.ds(e * LANES, LANES)] = p
            flat = lanes_i * LANES
            acc = plsc.load_gather(ps, [flat])
            for j in range(1, LANES):
                acc += plsc.load_gather(ps, [flat + j])
            outv[pl.ds(r * CHUNK + e0, LANES)] = 1.0 / (1.0 + jnp.exp(-acc))
            return carry

        lax.fori_loop(0, GROUPS, group_body, 0, unroll=False)

    fire(0, gidx_sa, gidx_da, srows_a, drows_a, sem_sa, sem_da)

    def pair_body(i, carry):
        r0 = 2 * i
        fire(r0 + 1, gidx_sb, gidx_db, srows_b, drows_b, sem_sb, sem_db)
        drain(gidx_sa, gidx_da, srows_a, drows_a, sem_sa, sem_da)
        compute(r0, srows_a, drows_a)
        fire(r0 + 2, gidx_sa, gidx_da, srows_a, drows_a, sem_sa, sem_da)
        drain(gidx_sb, gidx_db, srows_b, drows_b, sem_sb, sem_db)
        compute(r0 + 1, srows_b, drows_b)
        return carry

    lax.fori_loop(0, (ROUNDS - 1) // 2, pair_body, 0, unroll=False)
    drain(gidx_sa, gidx_da, srows_a, drows_a, sem_sa, sem_da)
    compute(ROUNDS - 1, srows_a, drows_a)

    pltpu.sync_copy(outv, out_hbm.at[pl.ds(base_w, EDGES_PER_W)])


@jax.jit
def _decode(z_pack, src, dst):
    mesh = plsc.VectorSubcoreMesh(core_axis_name="c", subcore_axis_name="s")
    return pl.kernel(
        _body,
        out_type=jax.ShapeDtypeStruct((N_EDGES,), jnp.float32),
        mesh=mesh,
        compiler_params=pltpu.CompilerParams(needs_layout_passes=False),
        scratch_types=[
            pltpu.VMEM_SHARED((N_ROWS, ROW), jnp.int32),
            pltpu.VMEM((ROUNDS, CHUNK), jnp.int32),
            pltpu.VMEM((ROUNDS, CHUNK), jnp.int32),
            pltpu.VMEM((CHUNK,), jnp.int32),
            pltpu.VMEM((CHUNK,), jnp.int32),
            pltpu.VMEM((CHUNK,), jnp.int32),
            pltpu.VMEM((CHUNK,), jnp.int32),
            pltpu.VMEM((CHUNK, ROW), jnp.int32),
            pltpu.VMEM((CHUNK, ROW), jnp.int32),
            pltpu.VMEM((CHUNK, ROW), jnp.int32),
            pltpu.VMEM((CHUNK, ROW), jnp.int32),
            pltpu.VMEM((LANES * LANES,), jnp.float32),
            pltpu.VMEM((EDGES_PER_W,), jnp.float32),
            pltpu.SemaphoreType.DMA,
            pltpu.SemaphoreType.DMA,
            pltpu.SemaphoreType.DMA,
            pltpu.SemaphoreType.DMA,
        ],
    )(z_pack, src, dst)


def kernel(z, edge_index):
    zb = z.astype(jnp.bfloat16)
    z_pack = lax.bitcast_convert_type(
        zb.reshape(N_NODES, WORDS, 2), jnp.int32).reshape(N_ROWS, ROW)
    src = edge_index[0].reshape(NW, ROUNDS, CHUNK)
    dst = edge_index[1].reshape(NW, ROUNDS, CHUNK)
    return _decode(z_pack, src, dst)


# speedup vs baseline: 1.0490x; 1.0490x over previous
import jax
import jax.numpy as jnp
from jax import lax
from jax.experimental import pallas as pl
from jax.experimental.pallas import tpu as pltpu
from jax.experimental.pallas import tpu_sc as plsc

N_NODES = 10000
DIM = 128
WORDS = DIM // 2
ROW = 2 * WORDS
N_ROWS = N_NODES // 2
N_EDGES = 320000
LANES = 16

_info = plsc.get_sparse_core_info()
NC = _info.num_cores
NS = _info.num_subcores
NW = NC * NS

EDGES_PER_W = N_EDGES // NW
CHUNK = 80
ROUNDS = EDGES_PER_W // CHUNK
GROUPS = CHUNK // LANES

STRIPE = (N_ROWS // NS) // 8 * 8
TAIL = N_ROWS - STRIPE * NS


def _body(z_hbm, src_hbm, dst_hbm, out_hbm,
          zspm, sidx, didx,
          gidx_sa, gidx_da, gidx_sb, gidx_db,
          srows_a, drows_a, srows_b, drows_b, ps, outv,
          sem_sa, sem_da, sem_sb, sem_db):
    sid = lax.axis_index("s")
    wid = sid * NC + lax.axis_index("c")
    base_w = wid * EDGES_PER_W

    lanes_i = lax.iota(jnp.int32, LANES)

    sbase = pl.multiple_of(sid * STRIPE, 8)
    pltpu.sync_copy(z_hbm.at[pl.ds(sbase, STRIPE), :],
                    zspm.at[pl.ds(sbase, STRIPE), :])
    pltpu.sync_copy(z_hbm.at[pl.ds(STRIPE * NS, TAIL), :],
                    zspm.at[pl.ds(STRIPE * NS, TAIL), :])

    pltpu.sync_copy(src_hbm.at[wid], sidx)
    pltpu.sync_copy(dst_hbm.at[wid], didx)

    plsc.subcore_barrier()

    def fire(r, gidx_s, gidx_d, srows, drows, sem_s, sem_d):
        for j in range(CHUNK // LANES):
            sl = pl.ds(j * LANES, LANES)
            gidx_s[sl] = lax.shift_right_logical(sidx[r, sl], 1)
            gidx_d[sl] = lax.shift_right_logical(didx[r, sl], 1)
        pltpu.async_copy(zspm.at[gidx_s], srows, sem_s)
        pltpu.async_copy(zspm.at[gidx_d], drows, sem_d)

    def drain(gidx_s, gidx_d, srows, drows, sem_s, sem_d):
        pltpu.make_async_copy(zspm.at[gidx_s], srows, sem_s).wait()
        pltpu.make_async_copy(zspm.at[gidx_d], drows, sem_d).wait()

    def compute(r, srows, drows):
        def group_body(g, carry):
            e0 = g * LANES
            soffv = (sidx[r, pl.ds(e0, LANES)] & 1) * WORDS
            doffv = (didx[r, pl.ds(e0, LANES)] & 1) * WORDS
            for e in range(LANES):
                soff = soffv[e]
                doff = doffv[e]
                pb = []
                for j in range(WORDS // LANES):
                    sw = srows[e0 + e, pl.ds(soff + j * LANES, LANES)]
                    dw = drows[e0 + e, pl.ds(doff + j * LANES, LANES)]
                    pb.append(plsc.bitcast(sw, jnp.bfloat16)
                              * plsc.bitcast(dw, jnp.bfloat16))
                q = (pb[0] + pb[1]) + (pb[2] + pb[3])
                qlo, qhi = plsc.unpack(q, format=plsc.PackFormat.INTERLEAVED)
                ps[pl.ds(e * LANES, LANES)] = qlo + qhi
            flat = lanes_i * LANES
            acc = plsc.load_gather(ps, [flat])
            for j in range(1, LANES):
                acc += plsc.load_gather(ps, [flat + j])
            outv[pl.ds(r * CHUNK + e0, LANES)] = 1.0 / (1.0 + jnp.exp(-acc))
            return carry

        lax.fori_loop(0, GROUPS, group_body, 0, unroll=False)

    fire(0, gidx_sa, gidx_da, srows_a, drows_a, sem_sa, sem_da)

    def pair_body(i, carry):
        r0 = 2 * i
        fire(r0 + 1, gidx_sb, gidx_db, srows_b, drows_b, sem_sb, sem_db)
        drain(gidx_sa, gidx_da, srows_a, drows_a, sem_sa, sem_da)
        compute(r0, srows_a, drows_a)
        fire(r0 + 2, gidx_sa, gidx_da, srows_a, drows_a, sem_sa, sem_da)
        drain(gidx_sb, gidx_db, srows_b, drows_b, sem_sb, sem_db)
        compute(r0 + 1, srows_b, drows_b)
        return carry

    lax.fori_loop(0, (ROUNDS - 1) // 2, pair_body, 0, unroll=False)
    drain(gidx_sa, gidx_da, srows_a, drows_a, sem_sa, sem_da)
    compute(ROUNDS - 1, srows_a, drows_a)

    pltpu.sync_copy(outv, out_hbm.at[pl.ds(base_w, EDGES_PER_W)])


@jax.jit
def _decode(z_pack, src, dst):
    mesh = plsc.VectorSubcoreMesh(core_axis_name="c", subcore_axis_name="s")
    return pl.kernel(
        _body,
        out_type=jax.ShapeDtypeStruct((N_EDGES,), jnp.float32),
        mesh=mesh,
        compiler_params=pltpu.CompilerParams(needs_layout_passes=False),
        scratch_types=[
            pltpu.VMEM_SHARED((N_ROWS, ROW), jnp.int32),
            pltpu.VMEM((ROUNDS, CHUNK), jnp.int32),
            pltpu.VMEM((ROUNDS, CHUNK), jnp.int32),
            pltpu.VMEM((CHUNK,), jnp.int32),
            pltpu.VMEM((CHUNK,), jnp.int32),
            pltpu.VMEM((CHUNK,), jnp.int32),
            pltpu.VMEM((CHUNK,), jnp.int32),
            pltpu.VMEM((CHUNK, ROW), jnp.int32),
            pltpu.VMEM((CHUNK, ROW), jnp.int32),
            pltpu.VMEM((CHUNK, ROW), jnp.int32),
            pltpu.VMEM((CHUNK, ROW), jnp.int32),
            pltpu.VMEM((LANES * LANES,), jnp.float32),
            pltpu.VMEM((EDGES_PER_W,), jnp.float32),
            pltpu.SemaphoreType.DMA,
            pltpu.SemaphoreType.DMA,
            pltpu.SemaphoreType.DMA,
            pltpu.SemaphoreType.DMA,
        ],
    )(z_pack, src, dst)


def kernel(z, edge_index):
    zb = z.astype(jnp.bfloat16)
    z_pack = lax.bitcast_convert_type(
        zb.reshape(N_NODES, WORDS, 2), jnp.int32).reshape(N_ROWS, ROW)
    src = edge_index[0].reshape(NW, ROUNDS, CHUNK)
    dst = edge_index[1].reshape(NW, ROUNDS, CHUNK)
    return _decode(z_pack, src, dst)


# speedup vs baseline: 1.4871x; 1.4177x over previous
import jax
import jax.numpy as jnp
from jax import lax
from jax.experimental import pallas as pl
from jax.experimental.pallas import tpu as pltpu
from jax.experimental.pallas import tpu_sc as plsc

N_NODES = 10000
DIM = 128
WORDS = DIM // 2
ROW = 2 * WORDS
N_ROWS = N_NODES // 2
N_EDGES = 320000
LANES = 16

_info = plsc.get_sparse_core_info()
NC = _info.num_cores
NS = _info.num_subcores
NW = NC * NS

EDGES_PER_W = N_EDGES // NW
CHUNK = 80
ROUNDS = EDGES_PER_W // CHUNK
GROUPS = CHUNK // LANES

STRIPE = (N_ROWS // NS) // 8 * 8
TAIL = N_ROWS - STRIPE * NS


def _body(z_hbm, src_hbm, dst_hbm, out_hbm,
          zspm, sidx, didx,
          gidx_sa, gidx_da, gidx_sb, gidx_db,
          srows_a, drows_a, srows_b, drows_b, ps, outv,
          sem_sa, sem_da, sem_sb, sem_db):
    sid = lax.axis_index("s")
    wid = sid * NC + lax.axis_index("c")
    base_w = wid * EDGES_PER_W

    lanes_i = lax.iota(jnp.int32, LANES)

    sbase = pl.multiple_of(sid * STRIPE, 8)
    pltpu.sync_copy(z_hbm.at[pl.ds(sbase, STRIPE), :],
                    zspm.at[pl.ds(sbase, STRIPE), :])
    pltpu.sync_copy(z_hbm.at[pl.ds(STRIPE * NS, TAIL), :],
                    zspm.at[pl.ds(STRIPE * NS, TAIL), :])

    pltpu.sync_copy(src_hbm.at[wid], sidx)
    pltpu.sync_copy(dst_hbm.at[wid], didx)

    plsc.subcore_barrier()

    def fire(r, gidx_s, gidx_d, srows, drows, sem_s, sem_d):
        for j in range(CHUNK // LANES):
            sl = pl.ds(j * LANES, LANES)
            gidx_s[sl] = lax.shift_right_logical(sidx[r, sl], 1)
            gidx_d[sl] = lax.shift_right_logical(didx[r, sl], 1)
        pltpu.async_copy(zspm.at[gidx_s], srows, sem_s)
        pltpu.async_copy(zspm.at[gidx_d], drows, sem_d)

    def drain(gidx_s, gidx_d, srows, drows, sem_s, sem_d):
        pltpu.make_async_copy(zspm.at[gidx_s], srows, sem_s).wait()
        pltpu.make_async_copy(zspm.at[gidx_d], drows, sem_d).wait()

    def compute(r, srows, drows):
        def group_body(g, carry):
            e0 = g * LANES
            soffv = (sidx[r, pl.ds(e0, LANES)] & 1) * WORDS
            doffv = (didx[r, pl.ds(e0, LANES)] & 1) * WORDS
            acc = jnp.zeros((LANES,), jnp.float32)
            for e in range(LANES):
                soff = soffv[e]
                doff = doffv[e]
                pb = []
                for j in range(WORDS // LANES):
                    sw = srows[e0 + e, pl.ds(soff + j * LANES, LANES)]
                    dw = drows[e0 + e, pl.ds(doff + j * LANES, LANES)]
                    pb.append(plsc.bitcast(sw, jnp.bfloat16)
                              * plsc.bitcast(dw, jnp.bfloat16))
                q = (pb[0] + pb[1]) + (pb[2] + pb[3])
                qlo, qhi = plsc.unpack(q, format=plsc.PackFormat.INTERLEAVED)
                s = jnp.sum(qlo + qhi)
                acc = jnp.where(lanes_i == e, s, acc)
            outv[pl.ds(r * CHUNK + e0, LANES)] = 1.0 / (1.0 + jnp.exp(-acc))
            return carry

        lax.fori_loop(0, GROUPS, group_body, 0, unroll=False)

    fire(0, gidx_sa, gidx_da, srows_a, drows_a, sem_sa, sem_da)

    def pair_body(i, carry):
        r0 = 2 * i
        fire(r0 + 1, gidx_sb, gidx_db, srows_b, drows_b, sem_sb, sem_db)
        drain(gidx_sa, gidx_da, srows_a, drows_a, sem_sa, sem_da)
        compute(r0, srows_a, drows_a)
        fire(r0 + 2, gidx_sa, gidx_da, srows_a, drows_a, sem_sa, sem_da)
        drain(gidx_sb, gidx_db, srows_b, drows_b, sem_sb, sem_db)
        compute(r0 + 1, srows_b, drows_b)
        return carry

    lax.fori_loop(0, (ROUNDS - 1) // 2, pair_body, 0, unroll=False)
    drain(gidx_sa, gidx_da, srows_a, drows_a, sem_sa, sem_da)
    compute(ROUNDS - 1, srows_a, drows_a)

    pltpu.sync_copy(outv, out_hbm.at[pl.ds(base_w, EDGES_PER_W)])


@jax.jit
def _decode(z_pack, src, dst):
    mesh = plsc.VectorSubcoreMesh(core_axis_name="c", subcore_axis_name="s")
    return pl.kernel(
        _body,
        out_type=jax.ShapeDtypeStruct((N_EDGES,), jnp.float32),
        mesh=mesh,
        compiler_params=pltpu.CompilerParams(needs_layout_passes=False),
        scratch_types=[
            pltpu.VMEM_SHARED((N_ROWS, ROW), jnp.int32),
            pltpu.VMEM((ROUNDS, CHUNK), jnp.int32),
            pltpu.VMEM((ROUNDS, CHUNK), jnp.int32),
            pltpu.VMEM((CHUNK,), jnp.int32),
            pltpu.VMEM((CHUNK,), jnp.int32),
            pltpu.VMEM((CHUNK,), jnp.int32),
            pltpu.VMEM((CHUNK,), jnp.int32),
            pltpu.VMEM((CHUNK, ROW), jnp.int32),
            pltpu.VMEM((CHUNK, ROW), jnp.int32),
            pltpu.VMEM((CHUNK, ROW), jnp.int32),
            pltpu.VMEM((CHUNK, ROW), jnp.int32),
            pltpu.VMEM((LANES * LANES,), jnp.float32),
            pltpu.VMEM((EDGES_PER_W,), jnp.float32),
            pltpu.SemaphoreType.DMA,
            pltpu.SemaphoreType.DMA,
            pltpu.SemaphoreType.DMA,
            pltpu.SemaphoreType.DMA,
        ],
    )(z_pack, src, dst)


def kernel(z, edge_index):
    zb = z.astype(jnp.bfloat16)
    z_pack = lax.bitcast_convert_type(
        zb.reshape(N_NODES, WORDS, 2), jnp.int32).reshape(N_ROWS, ROW)
    src = edge_index[0].reshape(NW, ROUNDS, CHUNK)
    dst = edge_index[1].reshape(NW, ROUNDS, CHUNK)
    return _decode(z_pack, src, dst)


# speedup vs baseline: 1.4884x; 1.0009x over previous
import jax
import jax.numpy as jnp
from jax import lax
from jax.experimental import pallas as pl
from jax.experimental.pallas import tpu as pltpu
from jax.experimental.pallas import tpu_sc as plsc

N_NODES = 10000
DIM = 128
WORDS = DIM // 2
ROW = 2 * WORDS
N_ROWS = N_NODES // 2
N_EDGES = 320000
LANES = 16

_info = plsc.get_sparse_core_info()
NC = _info.num_cores
NS = _info.num_subcores
NW = NC * NS

EDGES_PER_W = N_EDGES // NW
CHUNK = 80
ROUNDS = EDGES_PER_W // CHUNK
GROUPS = CHUNK // LANES

STRIPE = (N_ROWS // NS) // 8 * 8
TAIL = N_ROWS - STRIPE * NS


def _body(z_hbm, src_hbm, dst_hbm, out_hbm,
          zspm, sidx, didx,
          gidx_sa, gidx_da, gidx_sb, gidx_db,
          srows_a, drows_a, srows_b, drows_b, outv,
          sem_sa, sem_da, sem_sb, sem_db):
    sid = lax.axis_index("s")
    wid = sid * NC + lax.axis_index("c")
    base_w = wid * EDGES_PER_W

    lanes_i = lax.iota(jnp.int32, LANES)

    sbase = pl.multiple_of(sid * STRIPE, 8)
    pltpu.sync_copy(z_hbm.at[pl.ds(sbase, STRIPE), :],
                    zspm.at[pl.ds(sbase, STRIPE), :])
    pltpu.sync_copy(z_hbm.at[pl.ds(STRIPE * NS, TAIL), :],
                    zspm.at[pl.ds(STRIPE * NS, TAIL), :])

    pltpu.sync_copy(src_hbm.at[wid], sidx)
    pltpu.sync_copy(dst_hbm.at[wid], didx)

    plsc.subcore_barrier()

    def fire(r, gidx_s, gidx_d, srows, drows, sem_s, sem_d):
        for j in range(CHUNK // LANES):
            sl = pl.ds(j * LANES, LANES)
            gidx_s[sl] = lax.shift_right_logical(sidx[r, sl], 1)
            gidx_d[sl] = lax.shift_right_logical(didx[r, sl], 1)
        pltpu.async_copy(zspm.at[gidx_s], srows, sem_s)
        pltpu.async_copy(zspm.at[gidx_d], drows, sem_d)

    def drain(gidx_s, gidx_d, srows, drows, sem_s, sem_d):
        pltpu.make_async_copy(zspm.at[gidx_s], srows, sem_s).wait()
        pltpu.make_async_copy(zspm.at[gidx_d], drows, sem_d).wait()

    def compute(r, srows, drows):
        def group_body(g, carry):
            e0 = g * LANES
            soffv = (sidx[r, pl.ds(e0, LANES)] & 1) * WORDS
            doffv = (didx[r, pl.ds(e0, LANES)] & 1) * WORDS
            acc = jnp.zeros((LANES,), jnp.float32)
            for e in range(LANES):
                soff = soffv[e]
                doff = doffv[e]
                pb = []
                for j in range(WORDS // LANES):
                    sw = srows[e0 + e, pl.ds(soff + j * LANES, LANES)]
                    dw = drows[e0 + e, pl.ds(doff + j * LANES, LANES)]
                    pb.append(plsc.bitcast(sw, jnp.bfloat16)
                              * plsc.bitcast(dw, jnp.bfloat16))
                q = (pb[0] + pb[1]) + (pb[2] + pb[3])
                qlo, qhi = plsc.unpack(q, format=plsc.PackFormat.INTERLEAVED)
                s = jnp.sum(qlo + qhi)
                acc = jnp.where(lanes_i == e, s, acc)
            outv[pl.ds(r * CHUNK + e0, LANES)] = 1.0 / (1.0 + jnp.exp(-acc))
            return carry

        lax.fori_loop(0, GROUPS, group_body, 0, unroll=False)

    fire(0, gidx_sa, gidx_da, srows_a, drows_a, sem_sa, sem_da)

    def pair_body(i, carry):
        r0 = 2 * i
        fire(r0 + 1, gidx_sb, gidx_db, srows_b, drows_b, sem_sb, sem_db)
        drain(gidx_sa, gidx_da, srows_a, drows_a, sem_sa, sem_da)
        compute(r0, srows_a, drows_a)
        fire(r0 + 2, gidx_sa, gidx_da, srows_a, drows_a, sem_sa, sem_da)
        drain(gidx_sb, gidx_db, srows_b, drows_b, sem_sb, sem_db)
        compute(r0 + 1, srows_b, drows_b)
        return carry

    lax.fori_loop(0, (ROUNDS - 1) // 2, pair_body, 0, unroll=False)
    drain(gidx_sa, gidx_da, srows_a, drows_a, sem_sa, sem_da)
    compute(ROUNDS - 1, srows_a, drows_a)

    pltpu.sync_copy(outv, out_hbm.at[pl.ds(base_w, EDGES_PER_W)])


@jax.jit
def _decode(z_pack, src, dst):
    mesh = plsc.VectorSubcoreMesh(core_axis_name="c", subcore_axis_name="s")
    return pl.kernel(
        _body,
        out_type=jax.ShapeDtypeStruct((N_EDGES,), jnp.float32),
        mesh=mesh,
        compiler_params=pltpu.CompilerParams(needs_layout_passes=False),
        scratch_types=[
            pltpu.VMEM_SHARED((N_ROWS, ROW), jnp.int32),
            pltpu.VMEM((ROUNDS, CHUNK), jnp.int32),
            pltpu.VMEM((ROUNDS, CHUNK), jnp.int32),
            pltpu.VMEM((CHUNK,), jnp.int32),
            pltpu.VMEM((CHUNK,), jnp.int32),
            pltpu.VMEM((CHUNK,), jnp.int32),
            pltpu.VMEM((CHUNK,), jnp.int32),
            pltpu.VMEM((CHUNK, ROW), jnp.int32),
            pltpu.VMEM((CHUNK, ROW), jnp.int32),
            pltpu.VMEM((CHUNK, ROW), jnp.int32),
            pltpu.VMEM((CHUNK, ROW), jnp.int32),
            pltpu.VMEM((EDGES_PER_W,), jnp.float32),
            pltpu.SemaphoreType.DMA,
            pltpu.SemaphoreType.DMA,
            pltpu.SemaphoreType.DMA,
            pltpu.SemaphoreType.DMA,
        ],
    )(z_pack, src, dst)


def kernel(z, edge_index):
    zb = z.astype(jnp.bfloat16)
    z_pack = lax.bitcast_convert_type(
        zb.reshape(N_NODES, WORDS, 2), jnp.int32).reshape(N_ROWS, ROW)
    src = edge_index[0].reshape(NW, ROUNDS, CHUNK)
    dst = edge_index[1].reshape(NW, ROUNDS, CHUNK)
    return _decode(z_pack, src, dst)
